# baseline (speedup 1.0000x reference)
import functools

import jax
import jax.numpy as jnp
from jax import lax
from jax.experimental import pallas as pl
from jax.experimental.pallas import tpu as pltpu
from jax.experimental.pallas import tpu_sc as plsc

NC = 2
NS = 16
CH = 128

DST_W = 128
HI16 = -65536
NPAD = 10240


def _dense_pre(x, pos, W_in, b_in, W_val, W_src, W_dst, pos_W1,
               pos_W2, pos_b1, pos_b2, attn_W1, attn_b1, attn_W2, attn_b2):
  n = x.shape[0]

  def body(x_r, pos_r, W_in_r, b_in_r, W_val_r, W_src_r, W_dst_r, pW1_r,
           pos_W2_r, pos_b1_r, pos_b2_r, aW1_r, ab1_r, aW2_r, ab2_r,
           srctab_r, dsttab_r, s0_r, num0_r):
    xv = x_r[...]
    h = jax.nn.relu(jnp.dot(xv, W_in_r[...],
                            preferred_element_type=jnp.float32) + b_in_r[...])
    A = jnp.dot(h, W_src_r[...], preferred_element_type=jnp.float32)
    B = jnp.dot(h, W_dst_r[...], preferred_element_type=jnp.float32)
    V = jnp.dot(h, W_val_r[...], preferred_element_type=jnp.float32)
    PW = jnp.dot(pos_r[...], pW1_r[...], preferred_element_type=jnp.float32)
    aW1 = aW1_r[...]
    A1 = jnp.dot(A, aW1, preferred_element_type=jnp.float32)
    B1 = jnp.dot(B, aW1, preferred_element_type=jnp.float32)
    c = jax.nn.relu(jnp.dot(jax.nn.relu(pos_b1_r[...]), pos_W2_r[...],
                            preferred_element_type=jnp.float32) + pos_b2_r[...])
    c1 = jnp.dot(c, aW1, preferred_element_type=jnp.float32)
    hidl = jax.nn.relu(B1 - A1 + c1 + ab1_r[...])
    eel = jnp.exp(jax.nn.relu(
        jnp.dot(hidl, aW2_r[...], preferred_element_type=jnp.float32)
        + ab2_r[...]))
    apad = jnp.zeros((NPAD - xv.shape[0], 128), jnp.float32)
    s0_r[...] = jnp.concatenate([eel, apad], axis=0)
    num0_r[...] = jnp.concatenate([eel * (V + c), apad], axis=0)
    u0 = lax.bitcast_convert_type(jnp.concatenate([A1, PW], axis=1),
                                  jnp.int32) + 0x8000
    u1 = lax.bitcast_convert_type(V, jnp.int32) + 0x8000
    srctab_r[...] = lax.shift_right_logical(u0, 16) | (u1 & HI16)
    dsttab_r[...] = jnp.concatenate([B1, PW], axis=1)

  return pl.pallas_call(
      body,
      out_shape=(
          jax.ShapeDtypeStruct((n, 128), jnp.int32),
          jax.ShapeDtypeStruct((n, DST_W), jnp.float32),
          jax.ShapeDtypeStruct((NPAD, 128), jnp.float32),
          jax.ShapeDtypeStruct((NPAD, 128), jnp.float32),
      ),
  )(x, pos, W_in, b_in, W_val, W_src, W_dst, pos_W1,
    pos_W2, pos_b1, pos_b2, attn_W1, attn_b1, attn_W2, attn_b2)


def _sc_gather(srctab, dsttab, src, dst):
  e = src.shape[0]
  chunks = e // CH
  nw = NC * NS

  mesh = plsc.VectorSubcoreMesh(core_axis_name="c", subcore_axis_name="s",
                                num_cores=NC, num_subcores=NS)
  iters = (chunks + nw - 1) // nw

  @functools.partial(
      pl.kernel,
      out_type=(jax.ShapeDtypeStruct((e, 128), jnp.int32),
                jax.ShapeDtypeStruct((e, DST_W), jnp.float32)),
      mesh=mesh,
      scratch_types=[
          pltpu.VMEM((CH,), jnp.int32),
          pltpu.VMEM((CH,), jnp.int32),
          pltpu.VMEM((CH, 128), jnp.int32),
          pltpu.VMEM((CH, DST_W), jnp.float32),
          pltpu.SemaphoreType.DMA,
          pltpu.SemaphoreType.DMA,
      ],
  )
  def k(srctab_h, dsttab_h, src_h, dst_h, sg_h, dg_h,
        idx_s, idx_d, buf_s, buf_d, sem_s, sem_d):
    wid = lax.axis_index("s") * NC + lax.axis_index("c")

    def body(j, _):
      kk = wid + j * nw

      @pl.when(kk < chunks)
      def _():
        base = kk * CH
        pltpu.sync_copy(src_h.at[pl.ds(base, CH)], idx_s)
        pltpu.sync_copy(dst_h.at[pl.ds(base, CH)], idx_d)
        cp1 = pltpu.async_copy(srctab_h.at[idx_s], buf_s, sem_s)
        cp2 = pltpu.async_copy(dsttab_h.at[idx_d], buf_d, sem_d)
        cp1.wait()
        cp2.wait()
        pltpu.sync_copy(buf_s, sg_h.at[pl.ds(base, CH)])
        pltpu.sync_copy(buf_d, dg_h.at[pl.ds(base, CH)])

      return 0

    lax.fori_loop(0, iters, body, 0)

  return k(srctab, dsttab, src, dst)


def _edge_mlp(sg, dg, pos_b1, pos_W2, pos_b2,
              attn_W1, attn_b1, attn_W2, attn_b2):
  e = sg.shape[0]
  be = 2000
  grid = e // be

  def body(sg_r, dg_r, pb1_r, pW2_r, pb2_r, aW1_r, ab1_r, aW2_r, ab2_r,
           ee_r, nc_r):
    sgv = sg_r[...]
    p0 = lax.bitcast_convert_type(lax.shift_left(sgv, 16), jnp.float32)
    v_src = lax.bitcast_convert_type(sgv & HI16, jnp.float32)
    diff = dg_r[...] - p0
    h1 = jax.nn.relu(diff[:, 64:] + pb1_r[...])
    delta = jax.nn.relu(
        jnp.dot(h1, pW2_r[...], preferred_element_type=jnp.float32)
        + pb2_r[...])
    h2 = jax.nn.relu(
        diff[:, :64]
        + jnp.dot(delta, aW1_r[...], preferred_element_type=jnp.float32)
        + ab1_r[...])
    aa = jax.nn.relu(
        jnp.dot(h2, aW2_r[...], preferred_element_type=jnp.float32)
        + ab2_r[...])
    ee = jnp.exp(aa)
    ee_r[...] = ee
    nc_r[...] = ee * (v_src + delta)

  wspec = lambda shape: pl.BlockSpec(shape, lambda i: (0,) * len(shape))
  return pl.pallas_call(
      body,
      grid=(grid,),
      in_specs=[
          pl.BlockSpec((be, 128), lambda i: (i, 0)),
          pl.BlockSpec((be, DST_W), lambda i: (i, 0)),
          wspec((1, 64)), wspec((64, 128)), wspec((1, 128)),
          wspec((128, 64)), wspec((1, 64)), wspec((64, 128)), wspec((1, 128)),
      ],
      out_specs=(pl.BlockSpec((be, 128), lambda i: (i, 0)),
                 pl.BlockSpec((be, 128), lambda i: (i, 0))),
      out_shape=(jax.ShapeDtypeStruct((e, 128), jnp.float32),
                 jax.ShapeDtypeStruct((e, 128), jnp.float32)),
  )(sg, dg, pos_b1, pos_W2, pos_b2,
    attn_W1, attn_b1, attn_W2, attn_b2)


def _sc_scatter(ee, nc, dst, s0, num0):
  e = ee.shape[0]
  n = s0.shape[0]
  chunks = e // CH
  rows = n // NS
  iters = (chunks + NS - 1) // NS

  mesh = plsc.VectorSubcoreMesh(core_axis_name="c", subcore_axis_name="s",
                                num_cores=NC, num_subcores=NS)

  @functools.partial(
      pl.kernel,
      out_type=(jax.ShapeDtypeStruct((n, 128), jnp.float32),
                jax.ShapeDtypeStruct((n, 128), jnp.float32)),
      mesh=mesh,
      scratch_types=[
          pltpu.VMEM_SHARED((n, 128), jnp.float32),
          pltpu.VMEM((CH,), jnp.int32),
          pltpu.VMEM((CH, 128), jnp.float32),
      ],
  )
  def k(ee_h, nc_h, dst_h, s0_h, num0_h, s_out, num_out, acc, idx_d, buf):
    cid = lax.axis_index("c")
    sid = lax.axis_index("s")

    @pl.when(cid == 0)
    def _():
      pltpu.sync_copy(s0_h.at[pl.ds(sid * rows, rows)],
                      acc.at[pl.ds(sid * rows, rows)])

    @pl.when(cid == 1)
    def _():
      pltpu.sync_copy(num0_h.at[pl.ds(sid * rows, rows)],
                      acc.at[pl.ds(sid * rows, rows)])

    plsc.subcore_barrier()

    def body(j, _):
      kk = sid + j * NS

      @pl.when(kk < chunks)
      def _():
        base = kk * CH
        pltpu.sync_copy(dst_h.at[pl.ds(base, CH)], idx_d)

        @pl.when(cid == 0)
        def _():
          pltpu.sync_copy(ee_h.at[pl.ds(base, CH)], buf)

        @pl.when(cid == 1)
        def _():
          pltpu.sync_copy(nc_h.at[pl.ds(base, CH)], buf)

        pltpu.sync_copy(buf, acc.at[idx_d], add=True)

      return 0

    lax.fori_loop(0, iters, body, 0)
    plsc.subcore_barrier()

    @pl.when(cid == 0)
    def _():
      pltpu.sync_copy(acc.at[pl.ds(sid * rows, rows)],
                      s_out.at[pl.ds(sid * rows, rows)])

    @pl.when(cid == 1)
    def _():
      pltpu.sync_copy(acc.at[pl.ds(sid * rows, rows)],
                      num_out.at[pl.ds(sid * rows, rows)])

  return k(ee, nc, dst, s0, num0)


def _dense_post(s, num, x, W_out, b_out):
  n = x.shape[0]

  def body(s_r, num_r, x_r, W_r, b_r, o_r):
    agg = num_r[:n, :] / s_r[:n, :]
    o_r[...] = jax.nn.relu(
        jnp.dot(agg, W_r[...], preferred_element_type=jnp.float32)
        + b_r[...]) + x_r[...]

  return pl.pallas_call(
      body,
      out_shape=jax.ShapeDtypeStruct((n, 128), jnp.float32),
  )(s, num, x, W_out, b_out)


def kernel(x, pos, edge_index, W_in, b_in, W_out, b_out, W_val, W_src, W_dst,
           pos_W1, pos_b1, pos_W2, pos_b2, attn_W1, attn_b1, attn_W2, attn_b2):
  src = edge_index[0]
  dst = edge_index[1]
  b_in2 = b_in.reshape(1, -1)
  b_out2 = b_out.reshape(1, -1)
  pb1 = pos_b1.reshape(1, -1)
  pb2 = pos_b2.reshape(1, -1)
  ab1 = attn_b1.reshape(1, -1)
  ab2 = attn_b2.reshape(1, -1)

  srctab, dsttab, s0, num0 = _dense_pre(
      x, pos, W_in, b_in2, W_val, W_src, W_dst, pos_W1,
      pos_W2, pb1, pb2, attn_W1, ab1, attn_W2, ab2)
  parts = 4
  ep = src.shape[0] // parts
  srcs = [src[i * ep:(i + 1) * ep] for i in range(parts)]
  dsts = [dst[i * ep:(i + 1) * ep] for i in range(parts)]
  gathered = [_sc_gather(srctab, dsttab, srcs[i], dsts[i])
              for i in range(parts)]
  mlps = [_edge_mlp(sg, dg, pb1, pos_W2, pb2, attn_W1, ab1, attn_W2, ab2)
          for sg, dg in gathered]
  s, num = s0, num0
  for i in range(parts):
    s, num = _sc_scatter(mlps[i][0], mlps[i][1], dsts[i], s, num)
  return _dense_post(s, num, x, W_out, b_out2)


# speedup vs baseline: 1.1310x; 1.1310x over previous
import functools

import jax
import jax.numpy as jnp
from jax import lax
from jax.experimental import pallas as pl
from jax.experimental.pallas import tpu as pltpu
from jax.experimental.pallas import tpu_sc as plsc

NC = 2
NS = 16
CH = 128

DST_W = 128
HI16 = -65536
NPAD = 10240


def _dense_pre(x, pos, W_in, b_in, W_val, W_src, W_dst, pos_W1,
               pos_W2, pos_b1, pos_b2, attn_W1, attn_b1, attn_W2, attn_b2):
  n = x.shape[0]

  def body(x_r, pos_r, W_in_r, b_in_r, W_val_r, W_src_r, W_dst_r, pW1_r,
           pos_W2_r, pos_b1_r, pos_b2_r, aW1_r, ab1_r, aW2_r, ab2_r,
           srctab_r, dsttab_r, s0_r, num0_r):
    xv = x_r[...]
    h = jax.nn.relu(jnp.dot(xv, W_in_r[...],
                            preferred_element_type=jnp.float32) + b_in_r[...])
    A = jnp.dot(h, W_src_r[...], preferred_element_type=jnp.float32)
    B = jnp.dot(h, W_dst_r[...], preferred_element_type=jnp.float32)
    V = jnp.dot(h, W_val_r[...], preferred_element_type=jnp.float32)
    PW = jnp.dot(pos_r[...], pW1_r[...], preferred_element_type=jnp.float32)
    aW1 = aW1_r[...]
    A1 = jnp.dot(A, aW1, preferred_element_type=jnp.float32)
    B1 = jnp.dot(B, aW1, preferred_element_type=jnp.float32)
    c = jax.nn.relu(jnp.dot(jax.nn.relu(pos_b1_r[...]), pos_W2_r[...],
                            preferred_element_type=jnp.float32) + pos_b2_r[...])
    c1 = jnp.dot(c, aW1, preferred_element_type=jnp.float32)
    hidl = jax.nn.relu(B1 - A1 + c1 + ab1_r[...])
    eel = jnp.exp(jax.nn.relu(
        jnp.dot(hidl, aW2_r[...], preferred_element_type=jnp.float32)
        + ab2_r[...]))
    apad = jnp.zeros((NPAD - xv.shape[0], 128), jnp.float32)
    s0_r[...] = jnp.concatenate([eel, apad], axis=0)
    num0_r[...] = jnp.concatenate([eel * (V + c), apad], axis=0)
    u0 = lax.bitcast_convert_type(jnp.concatenate([A1, PW], axis=1),
                                  jnp.int32) + 0x8000
    u1 = lax.bitcast_convert_type(V, jnp.int32) + 0x8000
    srctab_r[...] = lax.shift_right_logical(u0, 16) | (u1 & HI16)
    dsttab_r[...] = jnp.concatenate([B1, PW], axis=1)

  return pl.pallas_call(
      body,
      out_shape=(
          jax.ShapeDtypeStruct((n, 128), jnp.int32),
          jax.ShapeDtypeStruct((n, DST_W), jnp.float32),
          jax.ShapeDtypeStruct((NPAD, 128), jnp.float32),
          jax.ShapeDtypeStruct((NPAD, 128), jnp.float32),
      ),
  )(x, pos, W_in, b_in, W_val, W_src, W_dst, pos_W1,
    pos_W2, pos_b1, pos_b2, attn_W1, attn_b1, attn_W2, attn_b2)


def _sc_gather(srctab, dsttab, src, dst):
  e = src.shape[0]
  chunks = e // CH
  nw = NC * NS
  base_it = chunks // nw
  rem = chunks - base_it * nw
  it = base_it + (1 if rem else 0)

  mesh = plsc.VectorSubcoreMesh(core_axis_name="c", subcore_axis_name="s",
                                num_cores=NC, num_subcores=NS)

  @functools.partial(
      pl.kernel,
      out_type=(jax.ShapeDtypeStruct((e, 128), jnp.int32),
                jax.ShapeDtypeStruct((e, DST_W), jnp.float32)),
      mesh=mesh,
      scratch_types=[
          pltpu.VMEM((2, CH), jnp.int32),
          pltpu.VMEM((2, CH), jnp.int32),
          pltpu.VMEM((2, CH, 128), jnp.int32),
          pltpu.VMEM((2, CH, DST_W), jnp.float32),
          pltpu.SemaphoreType.DMA,
          pltpu.SemaphoreType.DMA,
          pltpu.SemaphoreType.DMA,
          pltpu.SemaphoreType.DMA,
          pltpu.SemaphoreType.DMA,
          pltpu.SemaphoreType.DMA,
      ],
  )
  def k(srctab_h, dsttab_h, src_h, dst_h, sg_h, dg_h,
        idx_s, idx_d, buf_s, buf_d, si0, si1, sg0, sg1, sw0, sw1):
    wid = lax.axis_index("s") * NC + lax.axis_index("c")
    start = wid * base_it + jnp.minimum(wid, rem)
    my_it = base_it + jnp.where(wid < rem, 1, 0)
    sem_i = (si0, si1)
    sem_g = (sg0, sg1)
    sem_w = (sw0, sw1)

    def idx_issue(jj, b):
      pltpu.async_copy(src_h.at[pl.ds((start + jj) * CH, CH)],
                       idx_s.at[b], sem_i[b])
      pltpu.async_copy(dst_h.at[pl.ds((start + jj) * CH, CH)],
                       idx_d.at[b], sem_i[b])

    def idx_wait(b):
      pltpu.make_async_copy(src_h.at[pl.ds(0, CH)], idx_s.at[b],
                            sem_i[b]).wait()
      pltpu.make_async_copy(dst_h.at[pl.ds(0, CH)], idx_d.at[b],
                            sem_i[b]).wait()

    def gat_issue(b):
      pltpu.async_copy(srctab_h.at[idx_s.at[b]], buf_s.at[b], sem_g[b])
      pltpu.async_copy(dsttab_h.at[idx_d.at[b]], buf_d.at[b], sem_g[b])

    def gat_wait(b):
      pltpu.make_async_copy(srctab_h.at[idx_s.at[b]], buf_s.at[b],
                            sem_g[b]).wait()
      pltpu.make_async_copy(dsttab_h.at[idx_d.at[b]], buf_d.at[b],
                            sem_g[b]).wait()

    def wb_issue(jj, b):
      pltpu.async_copy(buf_s.at[b], sg_h.at[pl.ds((start + jj) * CH, CH)],
                       sem_w[b])
      pltpu.async_copy(buf_d.at[b], dg_h.at[pl.ds((start + jj) * CH, CH)],
                       sem_w[b])

    def wb_wait(b):
      pltpu.make_async_copy(buf_s.at[b], sg_h.at[pl.ds(0, CH)],
                            sem_w[b]).wait()
      pltpu.make_async_copy(buf_d.at[b], dg_h.at[pl.ds(0, CH)],
                            sem_w[b]).wait()

    idx_issue(0, 0)
    idx_wait(0)
    gat_issue(0)
    idx_issue(1, 1)

    def body(j2, _):
      for b in (0, 1):
        j = j2 * 2 + b
        nb = 1 - b

        @pl.when(j + 1 < my_it)
        def _():
          idx_wait(nb)

          @pl.when(j >= 1)
          def _():
            wb_wait(nb)

          gat_issue(nb)

        @pl.when(j < my_it)
        def _():
          gat_wait(b)
          wb_issue(j, b)

        @pl.when(j + 2 < my_it)
        def _():
          idx_issue(j + 2, b)

      return 0

    lax.fori_loop(0, (it + 1) // 2, body, 0)
    wb_wait(0)
    wb_wait(1)

  return k(srctab, dsttab, src, dst)


def _edge_mlp(sg, dg, pos_b1, pos_W2, pos_b2,
              attn_W1, attn_b1, attn_W2, attn_b2):
  e = sg.shape[0]
  be = 2000
  grid = e // be

  def body(sg_r, dg_r, pb1_r, pW2_r, pb2_r, aW1_r, ab1_r, aW2_r, ab2_r,
           ee_r, nc_r):
    sgv = sg_r[...]
    p0 = lax.bitcast_convert_type(lax.shift_left(sgv, 16), jnp.float32)
    v_src = lax.bitcast_convert_type(sgv & HI16, jnp.float32)
    diff = dg_r[...] - p0
    h1 = jax.nn.relu(diff[:, 64:] + pb1_r[...])
    delta = jax.nn.relu(
        jnp.dot(h1, pW2_r[...], preferred_element_type=jnp.float32)
        + pb2_r[...])
    h2 = jax.nn.relu(
        diff[:, :64]
        + jnp.dot(delta, aW1_r[...], preferred_element_type=jnp.float32)
        + ab1_r[...])
    aa = jax.nn.relu(
        jnp.dot(h2, aW2_r[...], preferred_element_type=jnp.float32)
        + ab2_r[...])
    ee = jnp.exp(aa)
    ee_r[...] = ee
    nc_r[...] = ee * (v_src + delta)

  wspec = lambda shape: pl.BlockSpec(shape, lambda i: (0,) * len(shape))
  return pl.pallas_call(
      body,
      grid=(grid,),
      in_specs=[
          pl.BlockSpec((be, 128), lambda i: (i, 0)),
          pl.BlockSpec((be, DST_W), lambda i: (i, 0)),
          wspec((1, 64)), wspec((64, 128)), wspec((1, 128)),
          wspec((128, 64)), wspec((1, 64)), wspec((64, 128)), wspec((1, 128)),
      ],
      out_specs=(pl.BlockSpec((be, 128), lambda i: (i, 0)),
                 pl.BlockSpec((be, 128), lambda i: (i, 0))),
      out_shape=(jax.ShapeDtypeStruct((e, 128), jnp.float32),
                 jax.ShapeDtypeStruct((e, 128), jnp.float32)),
  )(sg, dg, pos_b1, pos_W2, pos_b2,
    attn_W1, attn_b1, attn_W2, attn_b2)


def _sc_scatter(ee, nc, dst, s0, num0):
  e = ee.shape[0]
  n = s0.shape[0]
  chunks = e // CH
  rows = n // NS
  iters = (chunks + NS - 1) // NS

  mesh = plsc.VectorSubcoreMesh(core_axis_name="c", subcore_axis_name="s",
                                num_cores=NC, num_subcores=NS)

  @functools.partial(
      pl.kernel,
      out_type=(jax.ShapeDtypeStruct((n, 128), jnp.float32),
                jax.ShapeDtypeStruct((n, 128), jnp.float32)),
      mesh=mesh,
      scratch_types=[
          pltpu.VMEM_SHARED((n, 128), jnp.float32),
          pltpu.VMEM((CH,), jnp.int32),
          pltpu.VMEM((CH, 128), jnp.float32),
      ],
  )
  def k(ee_h, nc_h, dst_h, s0_h, num0_h, s_out, num_out, acc, idx_d, buf):
    cid = lax.axis_index("c")
    sid = lax.axis_index("s")

    @pl.when(cid == 0)
    def _():
      pltpu.sync_copy(s0_h.at[pl.ds(sid * rows, rows)],
                      acc.at[pl.ds(sid * rows, rows)])

    @pl.when(cid == 1)
    def _():
      pltpu.sync_copy(num0_h.at[pl.ds(sid * rows, rows)],
                      acc.at[pl.ds(sid * rows, rows)])

    plsc.subcore_barrier()

    def body(j, _):
      kk = sid + j * NS

      @pl.when(kk < chunks)
      def _():
        base = kk * CH
        pltpu.sync_copy(dst_h.at[pl.ds(base, CH)], idx_d)

        @pl.when(cid == 0)
        def _():
          pltpu.sync_copy(ee_h.at[pl.ds(base, CH)], buf)

        @pl.when(cid == 1)
        def _():
          pltpu.sync_copy(nc_h.at[pl.ds(base, CH)], buf)

        pltpu.sync_copy(buf, acc.at[idx_d], add=True)

      return 0

    lax.fori_loop(0, iters, body, 0)
    plsc.subcore_barrier()

    @pl.when(cid == 0)
    def _():
      pltpu.sync_copy(acc.at[pl.ds(sid * rows, rows)],
                      s_out.at[pl.ds(sid * rows, rows)])

    @pl.when(cid == 1)
    def _():
      pltpu.sync_copy(acc.at[pl.ds(sid * rows, rows)],
                      num_out.at[pl.ds(sid * rows, rows)])

  return k(ee, nc, dst, s0, num0)


def _dense_post(s, num, x, W_out, b_out):
  n = x.shape[0]

  def body(s_r, num_r, x_r, W_r, b_r, o_r):
    agg = num_r[:n, :] / s_r[:n, :]
    o_r[...] = jax.nn.relu(
        jnp.dot(agg, W_r[...], preferred_element_type=jnp.float32)
        + b_r[...]) + x_r[...]

  return pl.pallas_call(
      body,
      out_shape=jax.ShapeDtypeStruct((n, 128), jnp.float32),
  )(s, num, x, W_out, b_out)


def kernel(x, pos, edge_index, W_in, b_in, W_out, b_out, W_val, W_src, W_dst,
           pos_W1, pos_b1, pos_W2, pos_b2, attn_W1, attn_b1, attn_W2, attn_b2):
  src = edge_index[0]
  dst = edge_index[1]
  b_in2 = b_in.reshape(1, -1)
  b_out2 = b_out.reshape(1, -1)
  pb1 = pos_b1.reshape(1, -1)
  pb2 = pos_b2.reshape(1, -1)
  ab1 = attn_b1.reshape(1, -1)
  ab2 = attn_b2.reshape(1, -1)

  srctab, dsttab, s0, num0 = _dense_pre(
      x, pos, W_in, b_in2, W_val, W_src, W_dst, pos_W1,
      pos_W2, pb1, pb2, attn_W1, ab1, attn_W2, ab2)
  parts = 2
  ep = src.shape[0] // parts
  srcs = [src[i * ep:(i + 1) * ep] for i in range(parts)]
  dsts = [dst[i * ep:(i + 1) * ep] for i in range(parts)]
  gathered = [_sc_gather(srctab, dsttab, srcs[i], dsts[i])
              for i in range(parts)]
  mlps = [_edge_mlp(sg, dg, pb1, pos_W2, pb2, attn_W1, ab1, attn_W2, ab2)
          for sg, dg in gathered]
  s, num = s0, num0
  for i in range(parts):
    s, num = _sc_scatter(mlps[i][0], mlps[i][1], dsts[i], s, num)
  return _dense_post(s, num, x, W_out, b_out2)


# speedup vs baseline: 1.4479x; 1.2802x over previous
import functools

import jax
import jax.numpy as jnp
from jax import lax
from jax.experimental import pallas as pl
from jax.experimental.pallas import tpu as pltpu
from jax.experimental.pallas import tpu_sc as plsc

NC = 2
NS = 16
CH = 128

DST_W = 128
HI16 = -65536
NPAD = 10240


def _dense_pre(x, pos, W_in, b_in, W_val, W_src, W_dst, pos_W1,
               pos_W2, pos_b1, pos_b2, attn_W1, attn_b1, attn_W2, attn_b2):
  n = x.shape[0]

  def body(x_r, pos_r, W_in_r, b_in_r, W_val_r, W_src_r, W_dst_r, pW1_r,
           pos_W2_r, pos_b1_r, pos_b2_r, aW1_r, ab1_r, aW2_r, ab2_r,
           srctab_r, dsttab_r, s0_r, num0_r):
    xv = x_r[...]
    h = jax.nn.relu(jnp.dot(xv, W_in_r[...],
                            preferred_element_type=jnp.float32) + b_in_r[...])
    A = jnp.dot(h, W_src_r[...], preferred_element_type=jnp.float32)
    B = jnp.dot(h, W_dst_r[...], preferred_element_type=jnp.float32)
    V = jnp.dot(h, W_val_r[...], preferred_element_type=jnp.float32)
    PW = jnp.dot(pos_r[...], pW1_r[...], preferred_element_type=jnp.float32)
    aW1 = aW1_r[...]
    A1 = jnp.dot(A, aW1, preferred_element_type=jnp.float32)
    B1 = jnp.dot(B, aW1, preferred_element_type=jnp.float32)
    c = jax.nn.relu(jnp.dot(jax.nn.relu(pos_b1_r[...]), pos_W2_r[...],
                            preferred_element_type=jnp.float32) + pos_b2_r[...])
    c1 = jnp.dot(c, aW1, preferred_element_type=jnp.float32)
    hidl = jax.nn.relu(B1 - A1 + c1 + ab1_r[...])
    eel = jnp.exp(jax.nn.relu(
        jnp.dot(hidl, aW2_r[...], preferred_element_type=jnp.float32)
        + ab2_r[...]))
    apad = jnp.zeros((NPAD - xv.shape[0], 128), jnp.float32)
    s0_r[...] = jnp.concatenate([eel, apad], axis=0)
    num0_r[...] = jnp.concatenate([eel * (V + c), apad], axis=0)
    u0 = lax.bitcast_convert_type(jnp.concatenate([A1, PW], axis=1),
                                  jnp.int32) + 0x8000
    u1 = lax.bitcast_convert_type(V, jnp.int32) + 0x8000
    srctab_r[...] = lax.shift_right_logical(u0, 16) | (u1 & HI16)
    dsttab_r[...] = jnp.concatenate([B1, PW], axis=1)

  return pl.pallas_call(
      body,
      out_shape=(
          jax.ShapeDtypeStruct((n, 128), jnp.int32),
          jax.ShapeDtypeStruct((n, DST_W), jnp.float32),
          jax.ShapeDtypeStruct((NPAD, 128), jnp.float32),
          jax.ShapeDtypeStruct((NPAD, 128), jnp.float32),
      ),
  )(x, pos, W_in, b_in, W_val, W_src, W_dst, pos_W1,
    pos_W2, pos_b1, pos_b2, attn_W1, attn_b1, attn_W2, attn_b2)


def _sc_gather(srctab, dsttab, src, dst):
  e = src.shape[0]
  chunks = e // CH
  nw = NC * NS
  base_it = chunks // nw
  rem = chunks - base_it * nw
  it = base_it + (1 if rem else 0)

  mesh = plsc.VectorSubcoreMesh(core_axis_name="c", subcore_axis_name="s",
                                num_cores=NC, num_subcores=NS)

  @functools.partial(
      pl.kernel,
      out_type=(jax.ShapeDtypeStruct((e, 128), jnp.int32),
                jax.ShapeDtypeStruct((e, DST_W), jnp.float32)),
      mesh=mesh,
      scratch_types=[
          pltpu.VMEM((2, CH), jnp.int32),
          pltpu.VMEM((2, CH), jnp.int32),
          pltpu.VMEM((2, CH, 128), jnp.int32),
          pltpu.VMEM((2, CH, DST_W), jnp.float32),
          pltpu.SemaphoreType.DMA,
          pltpu.SemaphoreType.DMA,
          pltpu.SemaphoreType.DMA,
          pltpu.SemaphoreType.DMA,
          pltpu.SemaphoreType.DMA,
          pltpu.SemaphoreType.DMA,
      ],
  )
  def k(srctab_h, dsttab_h, src_h, dst_h, sg_h, dg_h,
        idx_s, idx_d, buf_s, buf_d, si0, si1, sg0, sg1, sw0, sw1):
    wid = lax.axis_index("s") * NC + lax.axis_index("c")
    start = wid * base_it + jnp.minimum(wid, rem)
    my_it = base_it + jnp.where(wid < rem, 1, 0)
    sem_i = (si0, si1)
    sem_g = (sg0, sg1)
    sem_w = (sw0, sw1)

    def idx_issue(jj, b):
      pltpu.async_copy(src_h.at[pl.ds((start + jj) * CH, CH)],
                       idx_s.at[b], sem_i[b])
      pltpu.async_copy(dst_h.at[pl.ds((start + jj) * CH, CH)],
                       idx_d.at[b], sem_i[b])

    def idx_wait(b):
      pltpu.make_async_copy(src_h.at[pl.ds(0, CH)], idx_s.at[b],
                            sem_i[b]).wait()
      pltpu.make_async_copy(dst_h.at[pl.ds(0, CH)], idx_d.at[b],
                            sem_i[b]).wait()

    def gat_issue(b):
      pltpu.async_copy(srctab_h.at[idx_s.at[b]], buf_s.at[b], sem_g[b])
      pltpu.async_copy(dsttab_h.at[idx_d.at[b]], buf_d.at[b], sem_g[b])

    def gat_wait(b):
      pltpu.make_async_copy(srctab_h.at[idx_s.at[b]], buf_s.at[b],
                            sem_g[b]).wait()
      pltpu.make_async_copy(dsttab_h.at[idx_d.at[b]], buf_d.at[b],
                            sem_g[b]).wait()

    def wb_issue(jj, b):
      pltpu.async_copy(buf_s.at[b], sg_h.at[pl.ds((start + jj) * CH, CH)],
                       sem_w[b])
      pltpu.async_copy(buf_d.at[b], dg_h.at[pl.ds((start + jj) * CH, CH)],
                       sem_w[b])

    def wb_wait(b):
      pltpu.make_async_copy(buf_s.at[b], sg_h.at[pl.ds(0, CH)],
                            sem_w[b]).wait()
      pltpu.make_async_copy(buf_d.at[b], dg_h.at[pl.ds(0, CH)],
                            sem_w[b]).wait()

    idx_issue(0, 0)
    idx_wait(0)
    gat_issue(0)
    idx_issue(1, 1)

    def body(j2, _):
      for b in (0, 1):
        j = j2 * 2 + b
        nb = 1 - b

        @pl.when(j + 1 < my_it)
        def _():
          idx_wait(nb)

          @pl.when(j >= 1)
          def _():
            wb_wait(nb)

          gat_issue(nb)

        @pl.when(j < my_it)
        def _():
          gat_wait(b)
          wb_issue(j, b)

        @pl.when(j + 2 < my_it)
        def _():
          idx_issue(j + 2, b)

      return 0

    lax.fori_loop(0, (it + 1) // 2, body, 0)
    wb_wait(0)
    wb_wait(1)

  return k(srctab, dsttab, src, dst)


def _edge_mlp(sg, dg, pos_b1, pos_W2, pos_b2,
              attn_W1, attn_b1, attn_W2, attn_b2):
  e = sg.shape[0]
  be = 2000
  grid = e // be

  def body(sg_r, dg_r, pb1_r, pW2_r, pb2_r, aW1_r, ab1_r, aW2_r, ab2_r,
           ee_r, nc_r):
    sgv = sg_r[...]
    p0 = lax.bitcast_convert_type(lax.shift_left(sgv, 16), jnp.float32)
    v_src = lax.bitcast_convert_type(sgv & HI16, jnp.float32)
    diff = dg_r[...] - p0
    h1 = jax.nn.relu(diff[:, 64:] + pb1_r[...])
    delta = jax.nn.relu(
        jnp.dot(h1, pW2_r[...], preferred_element_type=jnp.float32)
        + pb2_r[...])
    h2 = jax.nn.relu(
        diff[:, :64]
        + jnp.dot(delta, aW1_r[...], preferred_element_type=jnp.float32)
        + ab1_r[...])
    aa = jax.nn.relu(
        jnp.dot(h2, aW2_r[...], preferred_element_type=jnp.float32)
        + ab2_r[...])
    ee = jnp.exp(aa)
    ee_r[...] = ee
    nc_r[...] = ee * (v_src + delta)

  wspec = lambda shape: pl.BlockSpec(shape, lambda i: (0,) * len(shape))
  return pl.pallas_call(
      body,
      grid=(grid,),
      in_specs=[
          pl.BlockSpec((be, 128), lambda i: (i, 0)),
          pl.BlockSpec((be, DST_W), lambda i: (i, 0)),
          wspec((1, 64)), wspec((64, 128)), wspec((1, 128)),
          wspec((128, 64)), wspec((1, 64)), wspec((64, 128)), wspec((1, 128)),
      ],
      out_specs=(pl.BlockSpec((be, 128), lambda i: (i, 0)),
                 pl.BlockSpec((be, 128), lambda i: (i, 0))),
      out_shape=(jax.ShapeDtypeStruct((e, 128), jnp.float32),
                 jax.ShapeDtypeStruct((e, 128), jnp.float32)),
  )(sg, dg, pos_b1, pos_W2, pos_b2,
    attn_W1, attn_b1, attn_W2, attn_b2)


def _sc_scatter(ee, nc, dst, s0, num0):
  e = ee.shape[0]
  n = s0.shape[0]
  chs = 80
  chunks = e // chs
  rows = n // NS
  base_it = chunks // NS
  rem = chunks - base_it * NS
  it = base_it + (1 if rem else 0)

  mesh = plsc.VectorSubcoreMesh(core_axis_name="c", subcore_axis_name="s",
                                num_cores=NC, num_subcores=NS)

  @functools.partial(
      pl.kernel,
      out_type=(jax.ShapeDtypeStruct((n, 128), jnp.float32),
                jax.ShapeDtypeStruct((n, 128), jnp.float32)),
      mesh=mesh,
      scratch_types=[
          pltpu.VMEM_SHARED((n, 128), jnp.float32),
          pltpu.VMEM((3, chs), jnp.int32),
          pltpu.VMEM((3, chs, 128), jnp.float32),
          pltpu.SemaphoreType.DMA,
          pltpu.SemaphoreType.DMA,
          pltpu.SemaphoreType.DMA,
      ],
  )
  def k(ee_h, nc_h, dst_h, s0_h, num0_h, s_out, num_out, acc, idx_d, buf,
        sl0, sl1, sl2):
    cid = lax.axis_index("c")
    sid = lax.axis_index("s")
    start = sid * base_it + jnp.minimum(sid, rem)
    my_it = base_it + jnp.where(sid < rem, 1, 0)
    sem_l = (sl0, sl1, sl2)

    def load_issue(jj, b):
      base = (start + jj) * chs
      pltpu.async_copy(dst_h.at[pl.ds(base, chs)], idx_d.at[b], sem_l[b])

      @pl.when(cid == 0)
      def _():
        pltpu.async_copy(ee_h.at[pl.ds(base, chs)], buf.at[b], sem_l[b])

      @pl.when(cid == 1)
      def _():
        pltpu.async_copy(nc_h.at[pl.ds(base, chs)], buf.at[b], sem_l[b])

    def load_wait(b):
      pltpu.make_async_copy(dst_h.at[pl.ds(0, chs)], idx_d.at[b],
                            sem_l[b]).wait()
      pltpu.make_async_copy(ee_h.at[pl.ds(0, chs)], buf.at[b],
                            sem_l[b]).wait()

    @pl.when(cid == 0)
    def _():
      pltpu.sync_copy(s0_h.at[pl.ds(sid * rows, rows)],
                      acc.at[pl.ds(sid * rows, rows)])

    @pl.when(cid == 1)
    def _():
      pltpu.sync_copy(num0_h.at[pl.ds(sid * rows, rows)],
                      acc.at[pl.ds(sid * rows, rows)])

    load_issue(0, 0)
    load_issue(1, 1)
    load_issue(2, 2)
    plsc.subcore_barrier()

    def body(j3, _):
      for b in (0, 1, 2):
        j = j3 * 3 + b

        @pl.when(j < my_it)
        def _():
          load_wait(b)
          pltpu.sync_copy(buf.at[b], acc.at[idx_d.at[b]], add=True)

          @pl.when(j + 3 < my_it)
          def _():
            load_issue(j + 3, b)

      return 0

    lax.fori_loop(0, (it + 2) // 3, body, 0)
    plsc.subcore_barrier()

    @pl.when(cid == 0)
    def _():
      pltpu.sync_copy(acc.at[pl.ds(sid * rows, rows)],
                      s_out.at[pl.ds(sid * rows, rows)])

    @pl.when(cid == 1)
    def _():
      pltpu.sync_copy(acc.at[pl.ds(sid * rows, rows)],
                      num_out.at[pl.ds(sid * rows, rows)])

  return k(ee, nc, dst, s0, num0)


def _dense_post(s, num, x, W_out, b_out):
  n = x.shape[0]

  def body(s_r, num_r, x_r, W_r, b_r, o_r):
    agg = num_r[:n, :] / s_r[:n, :]
    o_r[...] = jax.nn.relu(
        jnp.dot(agg, W_r[...], preferred_element_type=jnp.float32)
        + b_r[...]) + x_r[...]

  return pl.pallas_call(
      body,
      out_shape=jax.ShapeDtypeStruct((n, 128), jnp.float32),
  )(s, num, x, W_out, b_out)


def kernel(x, pos, edge_index, W_in, b_in, W_out, b_out, W_val, W_src, W_dst,
           pos_W1, pos_b1, pos_W2, pos_b2, attn_W1, attn_b1, attn_W2, attn_b2):
  src = edge_index[0]
  dst = edge_index[1]
  b_in2 = b_in.reshape(1, -1)
  b_out2 = b_out.reshape(1, -1)
  pb1 = pos_b1.reshape(1, -1)
  pb2 = pos_b2.reshape(1, -1)
  ab1 = attn_b1.reshape(1, -1)
  ab2 = attn_b2.reshape(1, -1)

  srctab, dsttab, s0, num0 = _dense_pre(
      x, pos, W_in, b_in2, W_val, W_src, W_dst, pos_W1,
      pos_W2, pb1, pb2, attn_W1, ab1, attn_W2, ab2)
  parts = 2
  ep = src.shape[0] // parts
  srcs = [src[i * ep:(i + 1) * ep] for i in range(parts)]
  dsts = [dst[i * ep:(i + 1) * ep] for i in range(parts)]
  gathered = [_sc_gather(srctab, dsttab, srcs[i], dsts[i])
              for i in range(parts)]
  mlps = [_edge_mlp(sg, dg, pb1, pos_W2, pb2, attn_W1, ab1, attn_W2, ab2)
          for sg, dg in gathered]
  s, num = s0, num0
  for i in range(parts):
    s, num = _sc_scatter(mlps[i][0], mlps[i][1], dsts[i], s, num)
  return _dense_post(s, num, x, W_out, b_out2)
